# baseline (speedup 1.0000x reference)
import jax
import jax.numpy as jnp
from jax.experimental import pallas as pl
from jax.experimental.pallas import tpu as pltpu

_H = 64
_W = 64
_P = _H * _W
_HEAD = 16
_PF = _P + _HEAD + 2 * _W + _HEAD

_C0, _C1, _C2, _C3 = 16, 32, 64, 64


_IPS = 4


def _body(x_ref, w0_ref, b0_ref, w1_ref, b1_ref, w2_ref, b2_ref,
          o_ref, f0, f1, f2):
    for f in (f0, f1, f2):
        f[0:_HEAD, :] = jnp.zeros_like(f[0:_HEAD, :])
        f[_HEAD + _P:, :] = jnp.zeros_like(f[_HEAD + _P:, :])

    def store_replicas(f, act, c):
        for j in range(3):
            f[_HEAD - j:_HEAD - j + _P, j * c:(j + 1) * c] = act

    def layer(fin, w_ref, b_ref, kc):
        acc = None
        for i in range(3):
            base = i * _W + _HEAD
            a = fin[base:base + _P, :]
            b = w_ref[i * kc:(i + 1) * kc, :]
            part = jnp.dot(a, b, preferred_element_type=jnp.float32)
            acc = part if acc is None else acc + part
        return jnp.maximum(acc + b_ref[...], 0.0)

    for m in range(_IPS):
        xm = x_ref[m].astype(jnp.bfloat16).reshape(_C0, _P)
        store_replicas(f0, xm.T, _C0)
        a0 = layer(f0, w0_ref, b0_ref, 3 * _C0).astype(jnp.bfloat16)
        store_replicas(f1, a0, _C1)
        a1 = layer(f1, w1_ref, b1_ref, 3 * _C1).astype(jnp.bfloat16)
        store_replicas(f2, a1, _C2)
        a2 = layer(f2, w2_ref, b2_ref, 3 * _C2)
        hf = _H - 6
        a2c = a2.astype(jnp.bfloat16).T.reshape(_C3, _H, _W)[:, :hf, :hf]
        o_ref[m * _C3:(m + 1) * _C3] = a2c


def _prep_w(w):
    c_out, c_in = w.shape[0], w.shape[1]
    return jnp.transpose(w, (2, 3, 1, 0)).reshape(9 * c_in, c_out).astype(
        jnp.bfloat16)


def kernel(x, w0, b0, w1, b1, w2, b2):
    n = x.shape[0]

    hf = _H - 6
    out = pl.pallas_call(
        _body,
        out_shape=jax.ShapeDtypeStruct((n * _C3, hf, hf), jnp.bfloat16),
        grid=(n // _IPS,),
        in_specs=[
            pl.BlockSpec((_IPS, _C0, _H, _W), lambda i: (i, 0, 0, 0)),
            pl.BlockSpec((9 * _C0, _C1), lambda i: (0, 0)),
            pl.BlockSpec((1, _C1), lambda i: (0, 0)),
            pl.BlockSpec((9 * _C1, _C2), lambda i: (0, 0)),
            pl.BlockSpec((1, _C2), lambda i: (0, 0)),
            pl.BlockSpec((9 * _C2, _C3), lambda i: (0, 0)),
            pl.BlockSpec((1, _C3), lambda i: (0, 0)),
        ],
        out_specs=pl.BlockSpec((_IPS * _C3, hf, hf), lambda i: (i, 0, 0)),
        scratch_shapes=[
            pltpu.VMEM((_PF, 3 * _C0), jnp.bfloat16),
            pltpu.VMEM((_PF, 3 * _C1), jnp.bfloat16),
            pltpu.VMEM((_PF, 3 * _C2), jnp.bfloat16),
        ],
        compiler_params=pltpu.CompilerParams(
            dimension_semantics=("parallel",),
            vmem_limit_bytes=48 << 20),
    )(x, _prep_w(w0), b0.reshape(1, _C1), _prep_w(w1), b1.reshape(1, _C2),
      _prep_w(w2), b2.reshape(1, _C3))

    return out.reshape(n, _C3, hf, hf).astype(jnp.float32)


# speedup vs baseline: 1.1232x; 1.1232x over previous
import jax
import jax.numpy as jnp
from jax.experimental import pallas as pl
from jax.experimental.pallas import tpu as pltpu

_H = 64
_W = 64
_P = _H * _W
_HEAD = 16
_PF = _P + _HEAD + 2 * _W + _HEAD

_C0, _C1, _C2, _C3 = 16, 32, 64, 64


_IPS = 4


def _body(x_ref, w0_ref, b0_ref, w1_ref, b1_ref, w2_ref, b2_ref,
          o_ref, f0, f1, f2):
    @pl.when(pl.program_id(0) == 0)
    def _zero_pads():
        for f in (f0, f1, f2):
            f[0:_HEAD, :] = jnp.zeros_like(f[0:_HEAD, :])
            f[_HEAD + _P:, :] = jnp.zeros_like(f[_HEAD + _P:, :])

    def store_replicas(f, act, c):
        for j in range(3):
            f[_HEAD - j:_HEAD - j + _P, j * c:(j + 1) * c] = act

    def layer(fin, w_ref, b_ref, kc):
        acc = None
        for i in range(3):
            base = i * _W + _HEAD
            a = fin[base:base + _P, :]
            b = w_ref[i * kc:(i + 1) * kc, :]
            part = jnp.dot(a, b, preferred_element_type=jnp.float32)
            acc = part if acc is None else acc + part
        return jnp.maximum(acc + b_ref[...], 0.0)

    for m in range(_IPS):
        store_replicas(f0, x_ref[m].astype(jnp.bfloat16).T, _C0)
        a0 = layer(f0, w0_ref, b0_ref, 3 * _C0).astype(jnp.bfloat16)
        store_replicas(f1, a0, _C1)
        a1 = layer(f1, w1_ref, b1_ref, 3 * _C1).astype(jnp.bfloat16)
        store_replicas(f2, a1, _C2)
        a2 = layer(f2, w2_ref, b2_ref, 3 * _C2)
        hf = _H - 6
        o_ref[m] = a2.T.reshape(_C3, _H, _W)[:, :hf, :hf]


def _prep_w(w):
    c_out, c_in = w.shape[0], w.shape[1]
    return jnp.transpose(w, (2, 3, 1, 0)).reshape(9 * c_in, c_out).astype(
        jnp.bfloat16)


def kernel(x, w0, b0, w1, b1, w2, b2):
    n = x.shape[0]
    xf = x.reshape(n, _C0, _P)

    hf = _H - 6
    out = pl.pallas_call(
        _body,
        out_shape=jax.ShapeDtypeStruct((n, _C3, hf, hf), jnp.float32),
        grid=(n // _IPS,),
        in_specs=[
            pl.BlockSpec((_IPS, _C0, _P), lambda i: (i, 0, 0)),
            pl.BlockSpec((9 * _C0, _C1), lambda i: (0, 0)),
            pl.BlockSpec((1, _C1), lambda i: (0, 0)),
            pl.BlockSpec((9 * _C1, _C2), lambda i: (0, 0)),
            pl.BlockSpec((1, _C2), lambda i: (0, 0)),
            pl.BlockSpec((9 * _C2, _C3), lambda i: (0, 0)),
            pl.BlockSpec((1, _C3), lambda i: (0, 0)),
        ],
        out_specs=pl.BlockSpec((_IPS, _C3, hf, hf), lambda i: (i, 0, 0, 0)),
        scratch_shapes=[
            pltpu.VMEM((_PF, 3 * _C0), jnp.bfloat16),
            pltpu.VMEM((_PF, 3 * _C1), jnp.bfloat16),
            pltpu.VMEM((_PF, 3 * _C2), jnp.bfloat16),
        ],
        compiler_params=pltpu.CompilerParams(
            dimension_semantics=("parallel",),
            vmem_limit_bytes=48 << 20),
    )(xf, _prep_w(w0), b0.reshape(1, _C1), _prep_w(w1), b1.reshape(1, _C2),
      _prep_w(w2), b2.reshape(1, _C3))

    return out
